# Initial kernel scaffold; baseline (speedup 1.0000x reference)
import jax
import jax.numpy as jnp
from jax.experimental import pallas as pl


def kernel(input):
    raise NotImplementedError("write your pallas kernel here")


# speedup vs baseline: 2.1217x; 2.1217x over previous
import functools
import jax
import jax.numpy as jnp
from jax.experimental import pallas as pl


def _body(n, in_ref, out_ref):
    for r in range(n - 1, -1, -1):
        off = n * r - (r * (r - 1)) // 2
        base = off - r
        out_ref[:, pl.ds(base, n)] = in_ref[:, r, :]


def kernel(input):
    b, n, _ = input.shape
    m = (n * (n + 1)) // 2
    bb = 8 if b % 8 == 0 else 1
    grid = (b // bb,)
    return pl.pallas_call(
        functools.partial(_body, n),
        grid=grid,
        in_specs=[pl.BlockSpec((bb, n, n), lambda i: (i, 0, 0))],
        out_specs=pl.BlockSpec((bb, m), lambda i: (i, 0)),
        out_shape=jax.ShapeDtypeStruct((b, m), input.dtype),
    )(input)


# speedup vs baseline: 2.2301x; 1.0511x over previous
import functools
import jax
import jax.numpy as jnp
from jax.experimental import pallas as pl


def _body(n, in_ref, out_ref):
    for r in range(n):
        off = n * r - (r * (r - 1)) // 2
        out_ref[:, pl.ds(off, n - r)] = in_ref[:, r, pl.ds(r, n - r)]


def kernel(input):
    b, n, _ = input.shape
    m = (n * (n + 1)) // 2
    bb = 8 if b % 8 == 0 else 1
    grid = (b // bb,)
    return pl.pallas_call(
        functools.partial(_body, n),
        grid=grid,
        in_specs=[pl.BlockSpec((bb, n, n), lambda i: (i, 0, 0))],
        out_specs=pl.BlockSpec((bb, m), lambda i: (i, 0)),
        out_shape=jax.ShapeDtypeStruct((b, m), input.dtype),
    )(input)


# speedup vs baseline: 2.9900x; 1.3408x over previous
import functools
import jax
import jax.numpy as jnp
from jax.experimental import pallas as pl


def _off(n, r):
    return n * r - (r * (r - 1)) // 2


def _body_split(n, top_ref, bot_ref, out_ref):
    h = n // 2
    for r in range(h):
        out_ref[:, pl.ds(_off(n, r), n - r)] = top_ref[:, r, pl.ds(r, n - r)]
    for r in range(h, n):
        out_ref[:, pl.ds(_off(n, r), n - r)] = bot_ref[:, r - h, pl.ds(r - h, n - r)]


def _body_single(n, in_ref, out_ref):
    for r in range(n):
        out_ref[:, pl.ds(_off(n, r), n - r)] = in_ref[:, r, pl.ds(r, n - r)]


def kernel(input):
    b, n, _ = input.shape
    m = (n * (n + 1)) // 2
    h = n // 2
    bb = 16 if b % 16 == 0 else 1
    grid = (b // bb,)
    out_shape = jax.ShapeDtypeStruct((b, m), input.dtype)
    if h % 128 == 0:
        return pl.pallas_call(
            functools.partial(_body_split, n),
            grid=grid,
            in_specs=[
                pl.BlockSpec((bb, h, n), lambda i: (i, 0, 0)),
                pl.BlockSpec((bb, h, h), lambda i: (i, 1, 1)),
            ],
            out_specs=pl.BlockSpec((bb, m), lambda i: (i, 0)),
            out_shape=out_shape,
        )(input, input)
    return pl.pallas_call(
        functools.partial(_body_single, n),
        grid=grid,
        in_specs=[pl.BlockSpec((bb, n, n), lambda i: (i, 0, 0))],
        out_specs=pl.BlockSpec((bb, m), lambda i: (i, 0)),
        out_shape=out_shape,
    )(input)


# speedup vs baseline: 3.3981x; 1.1365x over previous
import functools
import jax
import jax.numpy as jnp
from jax.experimental import pallas as pl


def _off(n, r):
    return n * r - (r * (r - 1)) // 2


def _body_split(n, top_ref, bot_ref, out_ref):
    h = n // 2
    for r in range(h):
        out_ref[:, pl.ds(_off(n, r), n - r)] = top_ref[:, r, pl.ds(r, n - r)]
    for r in range(h, n):
        out_ref[:, pl.ds(_off(n, r), n - r)] = bot_ref[:, r - h, pl.ds(r - h, n - r)]


def _body_single(n, in_ref, out_ref):
    for r in range(n):
        out_ref[:, pl.ds(_off(n, r), n - r)] = in_ref[:, r, pl.ds(r, n - r)]


def kernel(input):
    b, n, _ = input.shape
    m = (n * (n + 1)) // 2
    h = n // 2
    bb = 32 if b % 32 == 0 else 1
    grid = (b // bb,)
    out_shape = jax.ShapeDtypeStruct((b, m), input.dtype)
    if h % 128 == 0:
        return pl.pallas_call(
            functools.partial(_body_split, n),
            grid=grid,
            in_specs=[
                pl.BlockSpec((bb, h, n), lambda i: (i, 0, 0)),
                pl.BlockSpec((bb, h, h), lambda i: (i, 1, 1)),
            ],
            out_specs=pl.BlockSpec((bb, m), lambda i: (i, 0)),
            out_shape=out_shape,
        )(input, input)
    return pl.pallas_call(
        functools.partial(_body_single, n),
        grid=grid,
        in_specs=[pl.BlockSpec((bb, n, n), lambda i: (i, 0, 0))],
        out_specs=pl.BlockSpec((bb, m), lambda i: (i, 0)),
        out_shape=out_shape,
    )(input)


# speedup vs baseline: 3.3996x; 1.0004x over previous
import functools
import jax
import jax.numpy as jnp
from jax.experimental import pallas as pl


def _off(n, r):
    return n * r - (r * (r - 1)) // 2


def _body_split(n, top_ref, bot_ref, out_ref):
    h = n // 2
    for r in range(h):
        out_ref[:, pl.ds(_off(n, r), n - r)] = top_ref[:, r, pl.ds(r, n - r)]
    for r in range(h, n):
        out_ref[:, pl.ds(_off(n, r), n - r)] = bot_ref[:, r - h, pl.ds(r - h, n - r)]


def _body_single(n, in_ref, out_ref):
    for r in range(n):
        out_ref[:, pl.ds(_off(n, r), n - r)] = in_ref[:, r, pl.ds(r, n - r)]


def kernel(input):
    b, n, _ = input.shape
    m = (n * (n + 1)) // 2
    h = n // 2
    bb = 64 if b % 64 == 0 else 1
    grid = (b // bb,)
    out_shape = jax.ShapeDtypeStruct((b, m), input.dtype)
    if h % 128 == 0:
        return pl.pallas_call(
            functools.partial(_body_split, n),
            grid=grid,
            in_specs=[
                pl.BlockSpec((bb, h, n), lambda i: (i, 0, 0)),
                pl.BlockSpec((bb, h, h), lambda i: (i, 1, 1)),
            ],
            out_specs=pl.BlockSpec((bb, m), lambda i: (i, 0)),
            out_shape=out_shape,
        )(input, input)
    return pl.pallas_call(
        functools.partial(_body_single, n),
        grid=grid,
        in_specs=[pl.BlockSpec((bb, n, n), lambda i: (i, 0, 0))],
        out_specs=pl.BlockSpec((bb, m), lambda i: (i, 0)),
        out_shape=out_shape,
    )(input)
